# Initial kernel scaffold; baseline (speedup 1.0000x reference)
import jax
import jax.numpy as jnp
from jax.experimental import pallas as pl


def kernel(x, idx, features):
    raise NotImplementedError("write your pallas kernel here")


# speedup vs baseline: 2.6039x; 2.6039x over previous
import functools

import jax
import jax.numpy as jnp
from jax import lax
from jax.experimental import pallas as pl
from jax.experimental.pallas import tpu as pltpu
from jax.experimental.pallas import tpu_sc as plsc

_T, _C, _D, _H, _W = 8, 32, 64, 64, 64
_V = _D * _H * _W
_N = 262144

_NC = 2
_NS = 16
_NW = _NC * _NS
_L = 16

_CH = 256
_NPW = _N // _NW
_NCHUNK = _NPW // _CH
_GROWS = 128
_G = 8 * _CH // _GROWS

_CORNER_OFFS = (0, 1, _W, _W + 1, _H * _W, _H * _W + 1, _H * _W + _W, _H * _W + _W + 1)


def _blend_body(fa_ref, fb_ref, w_ref, out_ref):
    out_ref[...] = fa_ref[...] * w_ref[0] + fb_ref[...] * w_ref[1]


def _blend(fa, fb, w):
    blk = 8192
    return pl.pallas_call(
        _blend_body,
        grid=(_V // blk,),
        in_specs=[
            pl.BlockSpec((_C, blk), lambda j: (0, j)),
            pl.BlockSpec((_C, blk), lambda j: (0, j)),
            pl.BlockSpec(memory_space=pltpu.SMEM),
        ],
        out_specs=pl.BlockSpec((_C, blk), lambda j: (0, j)),
        out_shape=jax.ShapeDtypeStruct((_C, _V), jnp.float32),
    )(fa, fb, w)


@functools.partial(
    pl.kernel,
    mesh=plsc.VectorSubcoreMesh(core_axis_name="c", subcore_axis_name="s"),
    out_type=jax.ShapeDtypeStruct((_N, _C), jnp.float32),
    compiler_params=pltpu.CompilerParams(use_tc_tiling_on_sc=False),
    scratch_types=[
        pltpu.VMEM((_CH,), jnp.float32),
        pltpu.VMEM((_CH,), jnp.float32),
        pltpu.VMEM((_CH,), jnp.float32),
        pltpu.VMEM((3 * _CH + _L,), jnp.float32),
        pltpu.VMEM((_G, _GROWS), jnp.int32),
        pltpu.VMEM((8 * _CH, _C), jnp.float32),
        pltpu.VMEM((_CH, _C), jnp.float32),
        pltpu.SemaphoreType.DMA,
    ],
)
def _sc_sample(xs_hbm, ys_hbm, zs_hbm, table_hbm, out_hbm,
               xs_v, ys_v, zs_v, t_v, idx_v, rows_v, out_v, sem):
    cid = lax.axis_index("c")
    sid = lax.axis_index("s")
    wid = sid * _NC + cid

    def chunk(ci, carry):
        base = pl.multiple_of(wid * _NPW + ci * _CH, _CH)
        pltpu.sync_copy(xs_hbm.at[pl.ds(base, _CH)], xs_v)
        pltpu.sync_copy(ys_hbm.at[pl.ds(base, _CH)], ys_v)
        pltpu.sync_copy(zs_hbm.at[pl.ds(base, _CH)], zs_v)

        for g in range(_CH // _L):
            sl = pl.ds(g * _L, _L)

            def axis_split(u_ref, size):
                u = jnp.minimum(jnp.maximum(u_ref[sl] * float(size) - 0.5, 0.0),
                                float(size - 1))
                i0 = jnp.minimum(u.astype(jnp.int32), size - 2)
                t = u - i0.astype(jnp.float32)
                return i0, t

            x0, tx = axis_split(xs_v, _W)
            y0, ty = axis_split(ys_v, _H)
            z0, tz = axis_split(zs_v, _D)
            t_v[pl.ds(g * _L, _L)] = tx
            t_v[pl.ds(_CH + g * _L, _L)] = ty
            t_v[pl.ds(2 * _CH + g * _L, _L)] = tz
            b = (z0 * _H + y0) * _W + x0
            for k, off in enumerate(_CORNER_OFFS):
                pos = k * _CH + g * _L
                idx_v[pos // _GROWS, pl.ds(pos % _GROWS, _L)] = b + off

        copies = [
            pltpu.async_copy(
                table_hbm.at[idx_v.at[j]],
                rows_v.at[pl.ds(j * _GROWS, _GROWS)],
                sem,
            )
            for j in range(_G)
        ]
        for cp in copies:
            cp.wait()

        def pt(p, acc):
            txv = jnp.full((_L,), t_v[pl.ds(p, _L)][0], jnp.float32)
            tyv = jnp.full((_L,), t_v[pl.ds(p + _CH, _L)][0], jnp.float32)
            tzv = jnp.full((_L,), t_v[pl.ds(p + 2 * _CH, _L)][0], jnp.float32)
            for cg in range(_C // _L):
                csl = pl.ds(cg * _L, _L)
                v = [rows_v[k * _CH + p, csl] for k in range(8)]
                cx00 = v[0] + txv * (v[1] - v[0])
                cx01 = v[2] + txv * (v[3] - v[2])
                cx10 = v[4] + txv * (v[5] - v[4])
                cx11 = v[6] + txv * (v[7] - v[6])
                c0 = cx00 + tyv * (cx01 - cx00)
                c1 = cx10 + tyv * (cx11 - cx10)
                out_v[p, csl] = c0 + tzv * (c1 - c0)
            return acc

        lax.fori_loop(0, _CH, pt, 0)
        pltpu.sync_copy(out_v, out_hbm.at[pl.ds(base, _CH)])
        return carry

    lax.fori_loop(0, _NCHUNK, chunk, 0)


def kernel(x, idx, features):
    num_t = features.shape[0]
    idx_val = idx.reshape(())
    t0 = jnp.clip(jnp.floor(idx_val).astype(jnp.int32), 0, num_t - 1)
    t1 = jnp.minimum(t0 + 1, num_t - 1)
    f = idx_val - t0.astype(jnp.float32)

    feats = features.reshape(num_t, _C, _V)
    fa = lax.dynamic_index_in_dim(feats, t0, 0, keepdims=False)
    fb = lax.dynamic_index_in_dim(feats, t1, 0, keepdims=False)
    w = jnp.stack([1.0 - f, f])

    table_cv = _blend(fa, fb, w)
    table = table_cv.T
    xt = x.T
    return _sc_sample(xt[0], xt[1], xt[2], table)


# speedup vs baseline: 2.8158x; 1.0814x over previous
import functools

import jax
import jax.numpy as jnp
from jax import lax
from jax.experimental import pallas as pl
from jax.experimental.pallas import tpu as pltpu
from jax.experimental.pallas import tpu_sc as plsc

_T, _C, _D, _H, _W = 8, 32, 64, 64, 64
_V = _D * _H * _W
_N = 262144

_NC = 2
_NS = 16
_NW = _NC * _NS
_L = 16

_CH = 256
_NPW = _N // _NW
_NCHUNK = _NPW // _CH
_GROWS = 128
_G = 8 * _CH // _GROWS

_CORNER_OFFS = (0, 1, _W, _W + 1, _H * _W, _H * _W + 1, _H * _W + _W, _H * _W + _W + 1)


def _blend_body(t_ref, fa_ref, fb_ref, w_ref, out_ref):
    blended = fa_ref[0] * w_ref[0] + fb_ref[0] * w_ref[1]
    out_ref[...] = jnp.transpose(blended)


def _blend(feats, tvec, w):
    blk = 8192
    return pl.pallas_call(
        _blend_body,
        grid_spec=pltpu.PrefetchScalarGridSpec(
            num_scalar_prefetch=1,
            grid=(_V // blk,),
            in_specs=[
                pl.BlockSpec((1, _C, blk), lambda j, t: (t[0], 0, j)),
                pl.BlockSpec((1, _C, blk), lambda j, t: (t[1], 0, j)),
                pl.BlockSpec(memory_space=pltpu.SMEM),
            ],
            out_specs=pl.BlockSpec((blk, _C), lambda j, t: (j, 0)),
        ),
        out_shape=jax.ShapeDtypeStruct((_V, _C), jnp.float32),
    )(tvec, feats, feats, w)


@functools.partial(
    pl.kernel,
    mesh=plsc.VectorSubcoreMesh(core_axis_name="c", subcore_axis_name="s"),
    out_type=jax.ShapeDtypeStruct((_N, _C), jnp.float32),
    compiler_params=pltpu.CompilerParams(use_tc_tiling_on_sc=False),
    scratch_types=[
        pltpu.VMEM((_CH,), jnp.float32),
        pltpu.VMEM((_CH,), jnp.float32),
        pltpu.VMEM((_CH,), jnp.float32),
        pltpu.VMEM((3 * _CH + _L,), jnp.float32),
        pltpu.VMEM((_G, _GROWS), jnp.int32),
        pltpu.VMEM((8 * _CH, _C), jnp.float32),
        pltpu.VMEM((_CH, _C), jnp.float32),
        pltpu.SemaphoreType.DMA,
    ],
)
def _sc_sample(xs_hbm, ys_hbm, zs_hbm, table_hbm, out_hbm,
               xs_v, ys_v, zs_v, t_v, idx_v, rows_v, out_v, sem):
    cid = lax.axis_index("c")
    sid = lax.axis_index("s")
    wid = sid * _NC + cid

    def chunk(ci, carry):
        base = pl.multiple_of(wid * _NPW + ci * _CH, _CH)
        pltpu.sync_copy(xs_hbm.at[pl.ds(base, _CH)], xs_v)
        pltpu.sync_copy(ys_hbm.at[pl.ds(base, _CH)], ys_v)
        pltpu.sync_copy(zs_hbm.at[pl.ds(base, _CH)], zs_v)

        for g in range(_CH // _L):
            sl = pl.ds(g * _L, _L)

            def axis_split(u_ref, size):
                u = jnp.minimum(jnp.maximum(u_ref[sl] * float(size) - 0.5, 0.0),
                                float(size - 1))
                i0 = jnp.minimum(u.astype(jnp.int32), size - 2)
                t = u - i0.astype(jnp.float32)
                return i0, t

            x0, tx = axis_split(xs_v, _W)
            y0, ty = axis_split(ys_v, _H)
            z0, tz = axis_split(zs_v, _D)
            t_v[pl.ds(g * _L, _L)] = tx
            t_v[pl.ds(_CH + g * _L, _L)] = ty
            t_v[pl.ds(2 * _CH + g * _L, _L)] = tz
            b = (z0 * _H + y0) * _W + x0
            for k, off in enumerate(_CORNER_OFFS):
                pos = k * _CH + g * _L
                idx_v[pos // _GROWS, pl.ds(pos % _GROWS, _L)] = b + off

        copies = [
            pltpu.async_copy(
                table_hbm.at[idx_v.at[j]],
                rows_v.at[pl.ds(j * _GROWS, _GROWS)],
                sem,
            )
            for j in range(_G)
        ]
        for cp in copies:
            cp.wait()

        def pt(p, acc):
            txv = jnp.full((_L,), t_v[pl.ds(p, _L)][0], jnp.float32)
            tyv = jnp.full((_L,), t_v[pl.ds(p + _CH, _L)][0], jnp.float32)
            tzv = jnp.full((_L,), t_v[pl.ds(p + 2 * _CH, _L)][0], jnp.float32)
            for cg in range(_C // _L):
                csl = pl.ds(cg * _L, _L)
                v = [rows_v[k * _CH + p, csl] for k in range(8)]
                cx00 = v[0] + txv * (v[1] - v[0])
                cx01 = v[2] + txv * (v[3] - v[2])
                cx10 = v[4] + txv * (v[5] - v[4])
                cx11 = v[6] + txv * (v[7] - v[6])
                c0 = cx00 + tyv * (cx01 - cx00)
                c1 = cx10 + tyv * (cx11 - cx10)
                out_v[p, csl] = c0 + tzv * (c1 - c0)
            return acc

        lax.fori_loop(0, _CH, pt, 0)
        pltpu.sync_copy(out_v, out_hbm.at[pl.ds(base, _CH)])
        return carry

    lax.fori_loop(0, _NCHUNK, chunk, 0)


def kernel(x, idx, features):
    num_t = features.shape[0]
    idx_val = idx.reshape(())
    t0 = jnp.clip(jnp.floor(idx_val).astype(jnp.int32), 0, num_t - 1)
    t1 = jnp.minimum(t0 + 1, num_t - 1)
    f = idx_val - t0.astype(jnp.float32)

    feats = features.reshape(num_t, _C, _V)
    w = jnp.stack([1.0 - f, f])
    tvec = jnp.stack([t0, t1])

    table = _blend(feats, tvec, w)
    xt = x.T
    return _sc_sample(xt[0], xt[1], xt[2], table)
